# baseline (speedup 1.0000x reference)
import functools

import jax
import jax.numpy as jnp
from jax.experimental import pallas as pl
from jax.experimental.pallas import tpu as pltpu

TOKENS = 32768
DIM = 768
N_EXPERTS = 8
BLOCK = 4096
NBUF = 4


def _gate_kernel(x_hbm, w_ref, w_out_ref, idx_out_ref):
    w = w_ref[...]

    def step(idx, x_blk_ref):
        (i,) = idx
        s = jax.lax.dot_general(
            x_blk_ref[...], w,
            dimension_numbers=(((1,), (1,)), ((), ())),
            preferred_element_type=jnp.float32)
        st = s.T
        m = jnp.max(st, axis=0, keepdims=True)
        denom = jnp.sum(jnp.exp(st - m), axis=0, keepdims=True)
        w_out_ref[pl.ds(i * BLOCK, BLOCK)] = (1.0 / denom).reshape(BLOCK)
        idx_out_ref[pl.ds(i * BLOCK, BLOCK)] = (
            jnp.argmax(st, axis=0).reshape(BLOCK).astype(jnp.int32))

    pltpu.emit_pipeline(
        step,
        grid=(TOKENS // BLOCK,),
        in_specs=[
            pl.BlockSpec((BLOCK, DIM), lambda i: (i, 0),
                         pipeline_mode=pl.Buffered(buffer_count=NBUF)),
        ],
        _explicit_indices=True,
    )(x_hbm)


@jax.jit
def kernel(x, weight):
    weights, indices = pl.pallas_call(
        _gate_kernel,
        in_specs=[
            pl.BlockSpec(memory_space=pltpu.HBM),
            pl.BlockSpec(memory_space=pltpu.VMEM),
        ],
        out_specs=[
            pl.BlockSpec(memory_space=pltpu.VMEM),
            pl.BlockSpec(memory_space=pltpu.VMEM),
        ],
        out_shape=[
            jax.ShapeDtypeStruct((TOKENS,), jnp.float32),
            jax.ShapeDtypeStruct((TOKENS,), jnp.int32),
        ],
    )(x, weight)
    return weights.reshape(TOKENS, 1), indices.reshape(TOKENS, 1)
